# baseline (speedup 1.0000x reference)
import functools

import jax
import jax.numpy as jnp
from jax import lax
from jax.experimental import pallas as pl
from jax.experimental.pallas import tpu as pltpu
from jax.experimental.pallas import tpu_sc as plsc

_PAIR = 2
_NC = 2
_NS = 16
_NW = _NC * _NS
_L = 16


def _logits_body(x_ref, Wg_ref, bg_ref, logits_ref):
    xb = x_ref[...].astype(jnp.bfloat16)
    logits_ref[...] = jax.lax.dot_general(
        xb, Wg_ref[...].astype(jnp.bfloat16), (((1,), (0,)), ((), ())),
        preferred_element_type=jnp.float32) + bg_ref[...]


def _make_route_sc(T, E):
    per_w = T // _NW
    ngroups = per_w // _L
    mesh = plsc.VectorSubcoreMesh(core_axis_name="c", subcore_axis_name="s")

    @functools.partial(
        pl.kernel, mesh=mesh,
        compiler_params=pltpu.CompilerParams(needs_layout_passes=False),
        out_type=jax.ShapeDtypeStruct((T * E,), jnp.float32),
        scratch_types=[
            pltpu.VMEM((per_w * E,), jnp.float32),
            pltpu.VMEM((per_w * E,), jnp.float32),
        ],
    )
    def _route(logits_hbm, comb_hbm, lbuf, cbuf):
        wid = lax.axis_index("s") * _NC + lax.axis_index("c")
        base = wid * per_w * E
        pltpu.sync_copy(logits_hbm.at[pl.ds(base, per_w * E)], lbuf)
        for g in range(ngroups):
            row = lax.iota(jnp.int32, _L) + g * _L
            lv = [plsc.load_gather(lbuf, [row * E + e]) for e in range(E)]
            m1 = lv[0]
            a1 = jnp.zeros((_L,), jnp.int32)
            for e in range(1, E):
                upd = lv[e] > m1
                m1 = jnp.where(upd, lv[e], m1)
                a1 = jnp.where(upd, e, a1)
            m2 = jnp.full((_L,), -jnp.inf, jnp.float32)
            a2 = jnp.zeros((_L,), jnp.int32)
            for e in range(E):
                val = jnp.where(a1 == e, m2, lv[e])
                upd = val > m2
                m2 = jnp.where(upd, val, m2)
                a2 = jnp.where(upd, e, a2)
            w1 = 1.0 / (1.0 + jnp.exp(m2 - m1))
            w2 = 1.0 - w1
            for e in range(E):
                ce = (jnp.where(a1 == e, w1, 0.0)
                      + jnp.where(a2 == e, w2, 0.0))
                plsc.store_scatter(cbuf, [row * E + e], ce)
        pltpu.sync_copy(cbuf, comb_hbm.at[pl.ds(base, per_w * E)])

    return _route


def _moe_body(x_ref, Ws_ref, bs_ref, Wr_ref, br_ref, comb_ref,
              out_ref, xbf_ref, acc_ref, accw_ref):
    p = pl.program_id(0)
    NP = pl.num_programs(0)

    dot = functools.partial(
        jax.lax.dot_general,
        dimension_numbers=(((1,), (0,)), ((), ())),
        preferred_element_type=jnp.float32)

    @pl.when(p == 0)
    def _init():
        xbf_ref[...] = x_ref[...].astype(jnp.bfloat16)
        accw_ref[...] = Ws_ref[0] + Ws_ref[1]

    @pl.when(p > 0)
    def _accw():
        accw_ref[...] += Ws_ref[0] + Ws_ref[1]

    comb = comb_ref[...]
    lane = jax.lax.broadcasted_iota(jnp.int32, comb.shape, 1)
    part = None
    for k in range(_PAIR):
        ye = dot(xbf_ref[...], Wr_ref[k].astype(jnp.bfloat16))
        e = p * _PAIR + k
        c_col = jnp.sum(jnp.where(lane == e, comb, 0.0), axis=1,
                        keepdims=True)
        yw = (ye * c_col).astype(jnp.bfloat16)
        part = yw if part is None else part + yw

    @pl.when(p == 0)
    def _():
        acc_ref[...] = part

    @pl.when((p > 0) & (p < NP - 1))
    def _():
        acc_ref[...] += part

    @pl.when(p == NP - 1)
    def _final():
        shared = dot(xbf_ref[...], accw_ref[...].astype(jnp.bfloat16))
        bsum = jnp.sum(bs_ref[...], axis=0, keepdims=True)
        rbias = dot(comb.astype(jnp.bfloat16),
                    br_ref[...].astype(jnp.bfloat16))
        out_ref[...] = ((acc_ref[...] + part).astype(jnp.float32)
                        + shared + bsum + rbias)


def kernel(x, Ws, bs, Wr, br, Wg, bg):
    b, s, h = x.shape
    E = Ws.shape[0]
    T = b * s
    x2 = x.reshape(T, h)
    bg2 = bg.reshape(1, E)
    npair = E // _PAIR

    logits = pl.pallas_call(
        _logits_body,
        in_specs=[
            pl.BlockSpec((T, h), lambda: (0, 0)),
            pl.BlockSpec((h, E), lambda: (0, 0)),
            pl.BlockSpec((1, E), lambda: (0, 0)),
        ],
        out_specs=pl.BlockSpec((T, E), lambda: (0, 0)),
        out_shape=jax.ShapeDtypeStruct((T, E), jnp.float32),
    )(x2, Wg, bg2)

    comb = _make_route_sc(T, E)(logits.reshape(T * E)).reshape(T, E)

    out = pl.pallas_call(
        _moe_body,
        grid=(npair,),
        in_specs=[
            pl.BlockSpec((T, h), lambda p: (0, 0)),
            pl.BlockSpec((_PAIR, h, h), lambda p: (p, 0, 0)),
            pl.BlockSpec((E, h), lambda p: (0, 0)),
            pl.BlockSpec((_PAIR, h, h), lambda p: (p, 0, 0)),
            pl.BlockSpec((E, h), lambda p: (0, 0)),
            pl.BlockSpec((T, E), lambda p: (0, 0)),
        ],
        out_specs=pl.BlockSpec((T, h), lambda p: (0, 0)),
        out_shape=jax.ShapeDtypeStruct((T, h), jnp.float32),
        scratch_shapes=[
            pltpu.VMEM((T, h), jnp.bfloat16),
            pltpu.VMEM((T, h), jnp.bfloat16),
            pltpu.VMEM((h, h), jnp.float32),
        ],
    )(x2, Ws, bs, Wr, br, comb)

    return out.reshape(b, s, h), logits


# speedup vs baseline: 1.5346x; 1.5346x over previous
import functools

import jax
import jax.numpy as jnp
from jax.experimental import pallas as pl
from jax.experimental.pallas import tpu as pltpu

_PAIR = 2


def _moe_body(x_ref, Ws_ref, bs_ref, Wr_ref, br_ref, Wg_ref, bg_ref,
              out_ref, logits_ref, xbf_ref, acc_ref, accw_ref, comb_ref):
    p = pl.program_id(0)
    NP = pl.num_programs(0)

    dot = functools.partial(
        jax.lax.dot_general,
        dimension_numbers=(((1,), (0,)), ((), ())),
        preferred_element_type=jnp.float32)

    @pl.when(p == 0)
    def _router():
        xbf_ref[...] = x_ref[...].astype(jnp.bfloat16)
        logits = dot(xbf_ref[...], Wg_ref[...].astype(jnp.bfloat16)) + bg_ref[...]
        logits_ref[...] = logits
        iota = jax.lax.broadcasted_iota(jnp.int32, logits.shape, 1)
        E = logits.shape[1]
        m1 = jnp.max(logits, axis=1, keepdims=True)
        a1 = jnp.min(jnp.where(logits == m1, iota, E), axis=1, keepdims=True)
        masked = jnp.where(iota == a1, -jnp.inf, logits)
        m2 = jnp.max(masked, axis=1, keepdims=True)
        a2 = jnp.min(jnp.where(masked == m2, iota, E), axis=1, keepdims=True)
        w1 = 1.0 / (1.0 + jnp.exp(m2 - m1))
        w2 = 1.0 - w1
        comb_ref[...] = (jnp.where(iota == a1, w1, 0.0)
                         + jnp.where(iota == a2, w2, 0.0))
        accw_ref[...] = Ws_ref[0] + Ws_ref[1]

    @pl.when(p > 0)
    def _accw():
        accw_ref[...] += Ws_ref[0] + Ws_ref[1]

    comb = comb_ref[...]
    lane = jax.lax.broadcasted_iota(jnp.int32, comb.shape, 1)
    part = None
    for k in range(_PAIR):
        ye = dot(xbf_ref[...], Wr_ref[k].astype(jnp.bfloat16))
        e = p * _PAIR + k
        c_col = jnp.sum(jnp.where(lane == e, comb, 0.0), axis=1,
                        keepdims=True)
        yw = (ye * c_col).astype(jnp.bfloat16)
        part = yw if part is None else part + yw

    @pl.when(p == 0)
    def _():
        acc_ref[...] = part

    @pl.when((p > 0) & (p < NP - 1))
    def _():
        acc_ref[...] += part

    @pl.when(p == NP - 1)
    def _final():
        shared = dot(xbf_ref[...], accw_ref[...].astype(jnp.bfloat16))
        bsum = jnp.sum(bs_ref[...], axis=0, keepdims=True)
        rbias = dot(comb_ref[...].astype(jnp.bfloat16),
                    br_ref[...].astype(jnp.bfloat16))
        out_ref[...] = ((acc_ref[...] + part).astype(jnp.float32)
                        + shared + bsum + rbias)


def kernel(x, Ws, bs, Wr, br, Wg, bg):
    b, s, h = x.shape
    E = Ws.shape[0]
    T = b * s
    x2 = x.reshape(T, h)
    bg2 = bg.reshape(1, E)
    npair = E // _PAIR

    def wmap(p):
        return (p, 0, 0)

    out, logits = pl.pallas_call(
        _moe_body,
        grid=(npair,),
        in_specs=[
            pl.BlockSpec((T, h), lambda p: (0, 0)),
            pl.BlockSpec((_PAIR, h, h), wmap),
            pl.BlockSpec((E, h), lambda p: (0, 0)),
            pl.BlockSpec((_PAIR, h, h), wmap),
            pl.BlockSpec((E, h), lambda p: (0, 0)),
            pl.BlockSpec((h, E), lambda p: (0, 0)),
            pl.BlockSpec((1, E), lambda p: (0, 0)),
        ],
        out_specs=[
            pl.BlockSpec((T, h), lambda p: (0, 0)),
            pl.BlockSpec((T, E), lambda p: (0, 0)),
        ],
        out_shape=[
            jax.ShapeDtypeStruct((T, h), jnp.float32),
            jax.ShapeDtypeStruct((T, E), jnp.float32),
        ],
        scratch_shapes=[
            pltpu.VMEM((T, h), jnp.bfloat16),
            pltpu.VMEM((T, h), jnp.bfloat16),
            pltpu.VMEM((h, h), jnp.float32),
            pltpu.VMEM((T, E), jnp.float32),
        ],
    )(x2, Ws, bs, Wr, br, Wg, bg2)

    return out.reshape(b, s, h), logits
